# baseline (speedup 1.0000x reference)
import functools

import jax
import jax.numpy as jnp
from jax import lax
from jax.experimental import pallas as pl
from jax.experimental.pallas import tpu as pltpu
from jax.experimental.pallas import tpu_sc as plsc

_N = 10000
_E = 320000
_D = 128
_NC = 2
_NS = 16
_EPW = _E // (_NC * _NS)
_B = 40
_ITERS = _EPW // _B
_ND = 6
_NPAD = 10112
_RPS = _NPAD // _NS
_ROWS_BLK = 10000


def _up_body(h_ref, w_ref, b_ref, o_ref):
    o_ref[...] = (
        jnp.dot(h_ref[...], w_ref[...], preferred_element_type=jnp.float32)
        + b_ref[...]
    )


def _up(h, w, b):
    return pl.pallas_call(
        _up_body,
        grid=(_N // _ROWS_BLK,),
        in_specs=[
            pl.BlockSpec((_ROWS_BLK, _D), lambda i: (i, 0)),
            pl.BlockSpec((_D, _D), lambda i: (0, 0)),
            pl.BlockSpec((1, _D), lambda i: (0, 0)),
        ],
        out_specs=pl.BlockSpec((_ROWS_BLK, _D), lambda i: (i, 0)),
        out_shape=jax.ShapeDtypeStruct((_N, _D), jnp.float32),
    )(h, w, b.reshape(1, _D))


def _mid_body(agg_ref, wd_ref, bd_ref, wu_ref, bu_ref, o_ref):
    a = agg_ref[0] + agg_ref[1]
    hm = jnp.dot(a, wd_ref[...], preferred_element_type=jnp.float32) + bd_ref[...]
    hm = hm * jax.nn.sigmoid(hm)
    o_ref[...] = (
        jnp.dot(hm, wu_ref[...], preferred_element_type=jnp.float32) + bu_ref[...]
    )


def _mid(agg, wd, bd, wu, bu):
    return pl.pallas_call(
        _mid_body,
        grid=(_N // _ROWS_BLK,),
        in_specs=[
            pl.BlockSpec((_NC, _ROWS_BLK, _D), lambda i: (0, i, 0)),
            pl.BlockSpec((_D, _D), lambda i: (0, 0)),
            pl.BlockSpec((1, _D), lambda i: (0, 0)),
            pl.BlockSpec((_D, _D), lambda i: (0, 0)),
            pl.BlockSpec((1, _D), lambda i: (0, 0)),
        ],
        out_specs=pl.BlockSpec((_ROWS_BLK, _D), lambda i: (i, 0)),
        out_shape=jax.ShapeDtypeStruct((_N, _D), jnp.float32),
    )(agg, wd, bd.reshape(1, _D), wu, bu.reshape(1, _D))


def _final_body(agg_ref, wd_ref, bd_ref, o_ref):
    a = agg_ref[0] + agg_ref[1]
    o_ref[...] = (
        jnp.dot(a, wd_ref[...], preferred_element_type=jnp.float32) + bd_ref[...]
    )


def _final(agg, wd, bd):
    return pl.pallas_call(
        _final_body,
        grid=(_N // _ROWS_BLK,),
        in_specs=[
            pl.BlockSpec((_NC, _ROWS_BLK, _D), lambda i: (0, i, 0)),
            pl.BlockSpec((_D, _D), lambda i: (0, 0)),
            pl.BlockSpec((1, _D), lambda i: (0, 0)),
        ],
        out_specs=pl.BlockSpec((_ROWS_BLK, _D), lambda i: (i, 0)),
        out_shape=jax.ShapeDtypeStruct((_N, _D), jnp.float32),
    )(agg, wd, bd.reshape(1, _D))


@functools.cache
def _make_sc_agg():
    @functools.partial(
        pl.kernel,
        out_type=jax.ShapeDtypeStruct((_NC, _NPAD, _D), jnp.float32),
        mesh=plsc.VectorSubcoreMesh(core_axis_name="c", subcore_axis_name="s"),
        scratch_types=[
            pltpu.VMEM_SHARED((_NPAD, _D), jnp.float32),
            pltpu.VMEM((_EPW,), jnp.int32),
            *[pltpu.VMEM((_B,), jnp.int32) for _ in range(_ND)],
            *[pltpu.VMEM((_B, _D), jnp.float32) for _ in range(_ND)],
            *[pltpu.SemaphoreType.DMA for _ in range(_ND)],
        ],
    )
    def _sc_agg(m_hbm, srcm_hbm, dstm_hbm, zeros_hbm,
                out_hbm, agg_s, src_a, *bufs):
        dsts = bufs[:_ND]
        rows = bufs[_ND:2 * _ND]
        sems = bufs[2 * _ND:3 * _ND]
        c = lax.axis_index("c")
        s = lax.axis_index("s")
        w = c * _NS + s
        pltpu.sync_copy(srcm_hbm.at[w], src_a)
        def prefetch(i, b):
            pltpu.async_copy(dstm_hbm.at[pl.ds(w * _EPW + i * _B, _B)],
                             dsts[b], sems[b])
            pltpu.async_copy(m_hbm.at[src_a.at[pl.ds(i * _B, _B)]],
                             rows[b], sems[b])

        def drain(b):
            pltpu.make_async_copy(dstm_hbm.at[pl.ds(0, _B)],
                                  dsts[b], sems[b]).wait()
            pltpu.make_async_copy(m_hbm.at[pl.ds(0, _B)],
                                  rows[b], sems[b]).wait()

        for b in range(_ND):
            prefetch(b, b)
        pltpu.sync_copy(zeros_hbm.at[pl.ds(s * _RPS, _RPS)],
                        agg_s.at[pl.ds(s * _RPS, _RPS)])
        plsc.subcore_barrier()

        def body(g, carry):
            for b in range(_ND):
                i = _ND * g + b
                drain(b)
                pltpu.sync_copy(rows[b], agg_s.at[dsts[b]], add=True)
                prefetch(i + _ND, b)
            return carry

        _MAIN = ((_ITERS - _ND) // _ND) * _ND
        lax.fori_loop(0, _MAIN // _ND, body, 0)
        for k in range(_MAIN, _ITERS):
            b = k % _ND
            drain(b)
            pltpu.sync_copy(rows[b], agg_s.at[dsts[b]], add=True)
            if k + _ND < _ITERS:
                prefetch(k + _ND, b)

        plsc.subcore_barrier()
        pltpu.sync_copy(agg_s.at[pl.ds(s * _RPS, _RPS)],
                        out_hbm.at[c, pl.ds(s * _RPS, _RPS)])

    return _sc_agg


def kernel(feat, edge_index,
           W_up0, b_up0, W_down0, b_down0,
           W_up1, b_up1, W_down1, b_down1,
           W_up2, b_up2, W_down2, b_down2):
    nw = _NC * _NS
    src_m = edge_index[0].reshape(nw, _EPW)
    dst_m = edge_index[1]
    zeros = jnp.zeros((_NPAD, _D), jnp.float32)
    sc_agg = _make_sc_agg()
    m = _up(feat, W_up0, b_up0)
    agg = sc_agg(m, src_m, dst_m, zeros)
    m = _mid(agg, W_down0, b_down0, W_up1, b_up1)
    agg = sc_agg(m, src_m, dst_m, zeros)
    m = _mid(agg, W_down1, b_down1, W_up2, b_up2)
    agg = sc_agg(m, src_m, dst_m, zeros)
    return _final(agg, W_down2, b_down2)
